# Initial kernel scaffold; baseline (speedup 1.0000x reference)
import jax
import jax.numpy as jnp
from jax.experimental import pallas as pl


def kernel(tokens, node_span_starts, node_span_ends, embed_table, pos_emb):
    raise NotImplementedError("write your pallas kernel here")


# speedup vs baseline: 3.3571x; 3.3571x over previous
import functools

import jax
import jax.numpy as jnp
from jax import lax
from jax.experimental import pallas as pl
from jax.experimental.pallas import tpu as pltpu
from jax.experimental.pallas import tpu_sc as plsc

BATCH = 1024
SEQ = 512
FEAT = 64
LANES = 16
IDX_CHUNK = 128


def kernel(tokens, node_span_starts, node_span_ends, embed_table, pos_emb):
    del node_span_starts, node_span_ends
    num_cores, num_subcores = 2, 16
    num_workers = num_cores * num_subcores
    rows_per_w = BATCH // num_workers

    mesh = plsc.VectorSubcoreMesh(core_axis_name="c", subcore_axis_name="s")

    @functools.partial(
        pl.kernel,
        mesh=mesh,
        compiler_params=pltpu.CompilerParams(use_tc_tiling_on_sc=False),
        out_type=jax.ShapeDtypeStruct((BATCH, SEQ, FEAT), jnp.float32),
        scratch_types=[
            pltpu.VMEM((SEQ,), jnp.int32),
            pltpu.VMEM((SEQ, FEAT), jnp.float32),
            pltpu.VMEM((SEQ, FEAT), jnp.float32),
            pltpu.SemaphoreType.DMA,
        ],
    )
    def body(tokens_hbm, table_hbm, pos_hbm, out_hbm, tok_v, pos_v, rows_v, sem):
        wid = lax.axis_index("s") * num_cores + lax.axis_index("c")
        base = wid * rows_per_w
        pltpu.sync_copy(pos_hbm.at[0], pos_v)

        def row_body(r, carry):
            row = base + r
            pltpu.sync_copy(tokens_hbm.at[row], tok_v)
            copies = [
                pltpu.async_copy(
                    table_hbm.at[tok_v.at[pl.ds(c * IDX_CHUNK, IDX_CHUNK)]],
                    rows_v.at[pl.ds(c * IDX_CHUNK, IDX_CHUNK)],
                    sem,
                )
                for c in range(SEQ // IDX_CHUNK)
            ]
            for cp in copies:
                cp.wait()

            def add_body(i, carry2):
                for j in range(FEAT // LANES):
                    sl = pl.ds(j * LANES, LANES)
                    rows_v[i, sl] = rows_v[i, sl] + pos_v[i, sl]
                return carry2

            lax.fori_loop(0, SEQ, add_body, 0)
            pltpu.sync_copy(rows_v, out_hbm.at[row])
            return carry

        lax.fori_loop(0, rows_per_w, row_body, 0)

    return body(tokens, embed_table, pos_emb)


# speedup vs baseline: 3.8885x; 1.1583x over previous
import functools

import jax
import jax.numpy as jnp
from jax import lax
from jax.experimental import pallas as pl
from jax.experimental.pallas import tpu as pltpu
from jax.experimental.pallas import tpu_sc as plsc

BATCH = 1024
SEQ = 512
FEAT = 64
LANES = 16
IDX_CHUNK = 128
HALF = SEQ // 2
NBUF = 4


def kernel(tokens, node_span_starts, node_span_ends, embed_table, pos_emb):
    del node_span_starts, node_span_ends
    num_cores, num_subcores = 2, 16
    num_workers = num_cores * num_subcores
    rows_per_w = BATCH // num_workers
    pairs = rows_per_w // 2

    mesh = plsc.VectorSubcoreMesh(core_axis_name="c", subcore_axis_name="s")

    @functools.partial(
        pl.kernel,
        mesh=mesh,
        compiler_params=pltpu.CompilerParams(use_tc_tiling_on_sc=False),
        out_type=jax.ShapeDtypeStruct((BATCH, SEQ, FEAT), jnp.float32),
        scratch_types=[
            pltpu.VMEM((SEQ, FEAT), jnp.float32),
            [pltpu.VMEM((HALF,), jnp.int32) for _ in range(NBUF)],
            [pltpu.VMEM((HALF, FEAT), jnp.float32) for _ in range(NBUF)],
            [pltpu.SemaphoreType.DMA for _ in range(NBUF)],
            [pltpu.SemaphoreType.DMA for _ in range(NBUF)],
        ],
    )
    def body(tokens_hbm, table_hbm, pos_hbm, out_hbm, pos_v, toks, bufs, gsems, wsems):
        wid = lax.axis_index("s") * num_cores + lax.axis_index("c")
        base = wid * rows_per_w
        pltpu.sync_copy(pos_hbm.at[0], pos_v)

        def fire_gather(b, row):
            off = (b % 2) * HALF
            pltpu.sync_copy(tokens_hbm.at[row, pl.ds(off, HALF)], toks[b])
            for c in range(HALF // IDX_CHUNK):
                sl = pl.ds(c * IDX_CHUNK, IDX_CHUNK)
                pltpu.async_copy(table_hbm.at[toks[b].at[sl]], bufs[b].at[sl], gsems[b])

        def wait_gather(b):
            pltpu.make_async_copy(table_hbm.at[pl.ds(0, HALF)], bufs[b], gsems[b]).wait()

        def wait_write(b):
            pltpu.make_async_copy(bufs[b], out_hbm.at[0, pl.ds(0, HALF)], wsems[b]).wait()

        def add_and_write(b, row):
            off = (b % 2) * HALF

            @plsc.parallel_loop(0, HALF, unroll=8)
            def _(i):
                for j in range(FEAT // LANES):
                    sl = pl.ds(j * LANES, LANES)
                    bufs[b][i, sl] = bufs[b][i, sl] + pos_v[off + i, sl]

            pltpu.async_copy(bufs[b], out_hbm.at[row, pl.ds(off, HALF)], wsems[b])

        for b in range(NBUF):
            fire_gather(b, base + b // 2)

        def pair_body(p, carry):
            row0 = base + 2 * p
            for b in range(NBUF):
                wait_gather(b)
                add_and_write(b, row0 + b // 2)
            nxt0 = row0 + 2
            for b in range(NBUF):
                wait_write(b)
                fire_gather(b, nxt0 + b // 2)
            return carry

        lax.fori_loop(0, pairs - 1, pair_body, 0)

        last0 = base + rows_per_w - 2
        for b in range(NBUF):
            wait_gather(b)
            add_and_write(b, last0 + b // 2)
        for b in range(NBUF):
            wait_write(b)

    return body(tokens, embed_table, pos_emb)


# speedup vs baseline: 4.1164x; 1.0586x over previous
import functools

import jax
import jax.numpy as jnp
from jax import lax
from jax.experimental import pallas as pl
from jax.experimental.pallas import tpu as pltpu
from jax.experimental.pallas import tpu_sc as plsc

BATCH = 1024
SEQ = 512
FEAT = 64
LANES = 16
CHUNK = 128
NBUF = 4
QPR = SEQ // CHUNK


def kernel(tokens, node_span_starts, node_span_ends, embed_table, pos_emb):
    del node_span_starts, node_span_ends
    num_cores, num_subcores = 2, 16
    num_workers = num_cores * num_subcores
    rows_per_w = BATCH // num_workers
    slab = rows_per_w * SEQ

    mesh = plsc.VectorSubcoreMesh(core_axis_name="c", subcore_axis_name="s")

    @functools.partial(
        pl.kernel,
        mesh=mesh,
        compiler_params=pltpu.CompilerParams(use_tc_tiling_on_sc=False),
        out_type=jax.ShapeDtypeStruct((BATCH * SEQ * FEAT,), jnp.float32),
        scratch_types=[
            pltpu.VMEM((slab,), jnp.int32),
            pltpu.VMEM((SEQ * FEAT,), jnp.float32),
            [pltpu.VMEM((CHUNK, FEAT), jnp.float32) for _ in range(NBUF)],
            [pltpu.VMEM((CHUNK * FEAT,), jnp.float32) for _ in range(NBUF)],
            [pltpu.SemaphoreType.DMA for _ in range(NBUF)],
            [pltpu.SemaphoreType.DMA for _ in range(NBUF)],
        ],
    )
    def body(tok_hbm, table_hbm, pos_hbm, out_hbm, tok_v, pos_v, gbufs, obufs,
             gsems, wsems):
        wid = lax.axis_index("s") * num_cores + lax.axis_index("c")
        base = wid * slab
        pltpu.sync_copy(tok_hbm.at[pl.ds(base, slab)], tok_v)
        pltpu.sync_copy(pos_hbm.at[pl.ds(0, SEQ * FEAT)], pos_v)

        def fire_gather(b, r):
            idx = tok_v.at[pl.ds(r * SEQ + b * CHUNK, CHUNK)]
            pltpu.async_copy(table_hbm.at[idx], gbufs[b], gsems[b])

        def wait_gather(b):
            pltpu.make_async_copy(table_hbm.at[pl.ds(0, CHUNK)], gbufs[b],
                                  gsems[b]).wait()

        def fire_write(b, r):
            dst = out_hbm.at[pl.ds(base * FEAT + (r * SEQ + b * CHUNK) * FEAT,
                                   CHUNK * FEAT)]
            pltpu.async_copy(obufs[b], dst, wsems[b])

        def wait_write(b):
            pltpu.make_async_copy(obufs[b], out_hbm.at[pl.ds(0, CHUNK * FEAT)],
                                  wsems[b]).wait()

        def add_chunk(b):
            @plsc.parallel_loop(0, CHUNK, unroll=8)
            def _(i):
                for j in range(FEAT // LANES):
                    sl = pl.ds(j * LANES, LANES)
                    obufs[b][pl.ds(i * FEAT + j * LANES, LANES)] = (
                        gbufs[b][i, sl]
                        + pos_v[pl.ds((b * CHUNK + i) * FEAT + j * LANES, LANES)])

        for b in range(NBUF):
            fire_gather(b, 0)
        for b in range(NBUF):
            wait_gather(b)
            add_chunk(b)
            fire_write(b, 0)
            fire_gather(b, 1)

        def row_body(r, carry):
            for b in range(NBUF):
                wait_gather(b)
                wait_write(b)
                add_chunk(b)
                fire_write(b, r)
                fire_gather(b, r + 1)
            return carry

        lax.fori_loop(1, rows_per_w - 1, row_body, 0)

        last = rows_per_w - 1
        for b in range(NBUF):
            wait_gather(b)
            wait_write(b)
            add_chunk(b)
            fire_write(b, last)
        for b in range(NBUF):
            wait_write(b)

    out = body(tokens.reshape(-1), embed_table, pos_emb.reshape(-1))
    return out.reshape(BATCH, SEQ, FEAT)
